# baseline (speedup 1.0000x reference)
import functools

import jax
import jax.numpy as jnp
from jax import lax
from jax.experimental import pallas as pl
from jax.experimental.pallas import tpu as pltpu
from jax.experimental.pallas import tpu_sc as plsc

_N_PROJ = 256
_NC, _NS, _LANES = 2, 16, 16
_NW = _NC * _NS


def _proj_body(x_ref, y_ref, th_ref, xo_ref, yo_ref):
    th = th_ref[...]
    nrm = jnp.maximum(jnp.sqrt(jnp.sum(th * th, axis=1, keepdims=True)), 1e-12)
    thn = th / nrm
    dn = (((1,), (1,)), ((), ()))
    xo_ref[...] = lax.dot_general(thn, x_ref[...], dn,
                                  preferred_element_type=jnp.float32)
    yo_ref[...] = lax.dot_general(thn, y_ref[...], dn,
                                  preferred_element_type=jnp.float32)


def _project(x, y, th_t):
    n, m = x.shape[0], y.shape[0]
    return pl.pallas_call(
        _proj_body,
        out_shape=[jax.ShapeDtypeStruct((_N_PROJ, n), jnp.float32),
                   jax.ShapeDtypeStruct((_N_PROJ, m), jnp.float32)],
    )(x, y, th_t)


def _vs(v):
    return plsc.sort_key_val(v, v)[0]


def _vsd(v, asc):
    return plsc.sort_key_val(v, v, descending=not asc)[0]


def _cmpx(a, b):
    return jnp.minimum(a, b), jnp.maximum(a, b)


def _cxd(a, b, asc):
    lo, hi = jnp.minimum(a, b), jnp.maximum(a, b)
    return (lo, hi) if asc else (hi, lo)


def _aligned_net(w):
    m = len(w)
    d = m // 2
    while d >= 1:
        for base in range(0, m, 2 * d):
            for o in range(d):
                w[base + o], w[base + o + d] = _cmpx(w[base + o], w[base + o + d])
        d //= 2
    return w


def _aligned_net_dir(w, asc):
    m = len(w)
    d = m // 2
    while d >= 1:
        for base in range(0, m, 2 * d):
            for o in range(d):
                w[base + o], w[base + o + d] = _cxd(w[base + o],
                                                    w[base + o + d], asc)
        d //= 2
    return w


def _merge_dir(c, asc):
    r = len(c) // 2
    for i in range(r):
        c[i], c[i + r] = _cxd(c[i], c[i + r], asc)
    if r >= 2:
        c[:r] = _aligned_net_dir(c[:r], asc)
        c[r:] = _aligned_net_dir(c[r:], asc)
    return [_vsd(v, asc) for v in c]


def _sort_block_dir(c):
    n = len(c)
    c = [_vsd(v, (k % 2 == 0)) for k, v in enumerate(c)]
    width = 1
    while width < n:
        for p0, p in enumerate(range(0, n, 2 * width)):
            c[p:p + 2 * width] = _merge_dir(c[p:p + 2 * width], (p0 % 2 == 0))
        width *= 2
    return c


def _ld(buf, u):
    return buf[pl.ds(u * 16, 16)]


def _st(buf, u, v):
    buf[pl.ds(u * 16, 16)] = v


def _mega_pass(buf, nv, off):
    @plsc.parallel_loop(0, nv // 32, unroll=1)
    def body(t):
        b = t * 32 + off
        w = [_ld(buf, b + k) for k in range(32)]
        w = _sort_block_dir(w)
        for k in range(32):
            _st(buf, b + k, w[k])


def _first_pass(buf, nv, lev, off):
    lrv = lev - 5

    @plsc.parallel_loop(0, nv // 16, unroll=1)
    def body(t):
        p = t >> (lrv - 3)
        io = (t << 3) & ((1 << lrv) - 1)
        a0 = (p << (lrv + 1)) + io + off
        b0 = (p << (lrv + 1)) + (2 << lrv) - 1 - io + off
        for k in range(8):
            va = _ld(buf, a0 + k)
            vb = lax.rev(_ld(buf, b0 - k), (0,))
            lo, hi = _cmpx(va, vb)
            _st(buf, a0 + k, lo)
            _st(buf, b0 - k, hi)


def _upper_pass(buf, nv, m, off):
    nb = 16 // m
    lnb = nb.bit_length() - 1

    @plsc.parallel_loop(0, nv // 16, unroll=1)
    def body(t):
        tb = t << lnb
        base = ((tb >> 4) << (4 + (m.bit_length() - 1))) + (tb & 15) + off
        for q in range(nb):
            w = [_ld(buf, base + q + k * 16) for k in range(m)]
            w = _aligned_net(w)
            for k in range(m):
                _st(buf, base + q + k * 16, w[k])


def _low_pass(buf, nv, off):
    @plsc.parallel_loop(0, nv // 16, unroll=1)
    def body(t):
        b = t * 16 + off
        w = [_ld(buf, b + k) for k in range(16)]
        w = _aligned_net(w)
        w = [_vs(v) for v in w]
        for k in range(16):
            _st(buf, b + k, w[k])


def _low_pass_interp(xbuf, ybuf, nv, m, yoff, acc):
    iota = lax.iota(jnp.int32, 16)
    even = (iota & 1) == 0

    @plsc.parallel_loop(0, nv // 16, unroll=1, carry=acc)
    def body(t, acc):
        b = t * 16
        w = [_ld(xbuf, b + k) for k in range(16)]
        w = _aligned_net(w)
        w = [_vs(v) for v in w]
        for k in range(16):
            j = (b + k) * 16 + iota
            ia = j >> 1
            ib = jnp.where(even, jnp.maximum(ia - 1, 0),
                           jnp.minimum(ia + 1, m - 1))
            ya = plsc.load_gather(ybuf, [yoff * 16 + ia])
            yb = plsc.load_gather(ybuf, [yoff * 16 + ib])
            d = w[k] - (0.75 * ya + 0.25 * yb)
            acc = acc + d * d
        return acc
    return body


def _sort_ref(buf, n, off):
    nv = n // 16
    _mega_pass(buf, nv, off)
    for lev in range(10, n.bit_length()):
        _first_pass(buf, nv, lev, off)
        _upper_pass(buf, nv, 1 << (lev - 9), off)
        _low_pass(buf, nv, off)


def _make_sc(n, m, ncols):
    cpw = ncols // _NW
    nvx, nvy = n // 16, m // 16
    mesh = plsc.VectorSubcoreMesh(core_axis_name="c", subcore_axis_name="s",
                                  num_cores=_NC, num_subcores=_NS)

    @functools.partial(
        pl.kernel,
        out_type=jax.ShapeDtypeStruct((_NW, _LANES), jnp.float32),
        mesh=mesh,
        scratch_types=[pltpu.VMEM((n,), jnp.float32),
                       pltpu.VMEM((2 * m,), jnp.float32),
                       pltpu.VMEM((_LANES,), jnp.float32),
                       pltpu.SemaphoreType.DMA,
                       pltpu.SemaphoreType.DMA],
        compiler_params=pltpu.CompilerParams(needs_layout_passes=False),
    )
    def sc(xp_hbm, yp_hbm, out_hbm, xbuf, ybuf, obuf, ysem, xsem):
        wid = lax.axis_index("s") * _NC + lax.axis_index("c")
        col0 = wid * cpw
        pltpu.async_copy(yp_hbm.at[col0], ybuf.at[pl.ds(0, m)], ysem)

        def col_body(c, acc):
            col = col0 + c
            yoff = (c & 1) * nvy
            ybase = yoff * 16
            pltpu.async_copy(xp_hbm.at[col], xbuf, xsem)
            pltpu.make_async_copy(yp_hbm.at[col],
                                  ybuf.at[pl.ds(ybase, m)], ysem).wait()

            @pl.when(c + 1 < cpw)
            def _():
                pltpu.async_copy(yp_hbm.at[col + 1],
                                 ybuf.at[pl.ds(m - ybase, m)],
                                 ysem)

            _sort_ref(ybuf, m, yoff)
            pltpu.make_async_copy(xp_hbm.at[col], xbuf, xsem).wait()
            _mega_pass(xbuf, nvx, 0)
            for lev in range(10, n.bit_length()):
                _first_pass(xbuf, nvx, lev, 0)
                _upper_pass(xbuf, nvx, 1 << (lev - 9), 0)
                if lev < n.bit_length() - 1:
                    _low_pass(xbuf, nvx, 0)
            return _low_pass_interp(xbuf, ybuf, nvx, m, yoff, acc)

        acc = lax.fori_loop(0, cpw, col_body, jnp.zeros((16,), jnp.float32))
        obuf[...] = acc
        pltpu.sync_copy(obuf, out_hbm.at[wid])

    return sc


def kernel(x, y):
    n, d = x.shape
    m = y.shape[0]
    assert n == 2 * m, "kernel specialized to N == 2*M"
    theta = jax.random.normal(jax.random.key(42), (d, _N_PROJ), dtype=x.dtype)
    th_t = theta.T
    xp, yp = _project(x, y, th_t)
    parts = _make_sc(n, m, _N_PROJ)(xp, yp)
    swd2 = jnp.sum(parts) / (n * _N_PROJ)
    return jnp.maximum(jnp.sqrt(swd2), jnp.float32(1e-8))


# speedup vs baseline: 1.0071x; 1.0071x over previous
import functools

import jax
import jax.numpy as jnp
from jax import lax
from jax.experimental import pallas as pl
from jax.experimental.pallas import tpu as pltpu
from jax.experimental.pallas import tpu_sc as plsc

_N_PROJ = 256
_NC, _NS, _LANES = 2, 16, 16
_NW = _NC * _NS


def _proj_body(x_ref, y_ref, th_ref, xo_ref, yo_ref):
    th = th_ref[...]
    nrm = jnp.maximum(jnp.sqrt(jnp.sum(th * th, axis=1, keepdims=True)), 1e-12)
    thn = th / nrm
    dn = (((1,), (1,)), ((), ()))
    xo_ref[...] = lax.dot_general(thn, x_ref[...], dn,
                                  preferred_element_type=jnp.float32)
    yo_ref[...] = lax.dot_general(thn, y_ref[...], dn,
                                  preferred_element_type=jnp.float32)


def _project(x, y, th_t):
    n, m = x.shape[0], y.shape[0]
    return pl.pallas_call(
        _proj_body,
        out_shape=[jax.ShapeDtypeStruct((_N_PROJ, n), jnp.float32),
                   jax.ShapeDtypeStruct((_N_PROJ, m), jnp.float32)],
    )(x, y, th_t)


def _vs(v):
    return plsc.sort_key_val(v, v)[0]


def _vsd(v, asc):
    return plsc.sort_key_val(v, v, descending=not asc)[0]


def _cmpx(a, b):
    return jnp.minimum(a, b), jnp.maximum(a, b)


def _cxd(a, b, asc):
    lo, hi = jnp.minimum(a, b), jnp.maximum(a, b)
    return (lo, hi) if asc else (hi, lo)


def _aligned_net(w):
    m = len(w)
    d = m // 2
    while d >= 1:
        for base in range(0, m, 2 * d):
            for o in range(d):
                w[base + o], w[base + o + d] = _cmpx(w[base + o], w[base + o + d])
        d //= 2
    return w


def _aligned_net_dir(w, asc):
    m = len(w)
    d = m // 2
    while d >= 1:
        for base in range(0, m, 2 * d):
            for o in range(d):
                w[base + o], w[base + o + d] = _cxd(w[base + o],
                                                    w[base + o + d], asc)
        d //= 2
    return w


def _merge_dir(c, asc):
    r = len(c) // 2
    for i in range(r):
        c[i], c[i + r] = _cxd(c[i], c[i + r], asc)
    if r >= 2:
        c[:r] = _aligned_net_dir(c[:r], asc)
        c[r:] = _aligned_net_dir(c[r:], asc)
    return [_vsd(v, asc) for v in c]


def _sort_block_dir(c):
    n = len(c)
    c = [_vsd(v, (k % 2 == 0)) for k, v in enumerate(c)]
    width = 1
    while width < n:
        for p0, p in enumerate(range(0, n, 2 * width)):
            c[p:p + 2 * width] = _merge_dir(c[p:p + 2 * width], (p0 % 2 == 0))
        width *= 2
    return c


def _ld(buf, u):
    return buf[pl.ds(u * 16, 16)]


def _st(buf, u, v):
    buf[pl.ds(u * 16, 16)] = v


def _mega_pass(buf, nv, off):
    @plsc.parallel_loop(0, nv // 32, unroll=1)
    def body(t):
        b = t * 32 + off
        w = [_ld(buf, b + k) for k in range(32)]
        w = _sort_block_dir(w)
        for k in range(32):
            _st(buf, b + k, w[k])


def _first_pass(buf, nv, lev, off):
    lrv = lev - 5

    @plsc.parallel_loop(0, nv // 16, unroll=1)
    def body(t):
        p = t >> (lrv - 3)
        io = (t << 3) & ((1 << lrv) - 1)
        a0 = (p << (lrv + 1)) + io + off
        b0 = (p << (lrv + 1)) + (2 << lrv) - 1 - io + off
        for k in range(8):
            va = _ld(buf, a0 + k)
            vb = lax.rev(_ld(buf, b0 - k), (0,))
            lo, hi = _cmpx(va, vb)
            _st(buf, a0 + k, lo)
            _st(buf, b0 - k, lax.rev(hi, (0,)))


def _upper_pass(buf, nv, m, off):
    nb = 16 // m
    lnb = nb.bit_length() - 1

    @plsc.parallel_loop(0, nv // 16, unroll=1)
    def body(t):
        tb = t << lnb
        base = ((tb >> 4) << (4 + (m.bit_length() - 1))) + (tb & 15) + off
        for q in range(nb):
            w = [_ld(buf, base + q + k * 16) for k in range(m)]
            w = _aligned_net(w)
            for k in range(m):
                _st(buf, base + q + k * 16, w[k])


def _low_pass(buf, nv, off):
    @plsc.parallel_loop(0, nv // 16, unroll=1)
    def body(t):
        b = t * 16 + off
        w = [_ld(buf, b + k) for k in range(16)]
        w = _aligned_net(w)
        w = [_vs(v) for v in w]
        for k in range(16):
            _st(buf, b + k, w[k])


def _low_pass_interp(xbuf, ybuf, nv, m, yoff, acc):
    iota = lax.iota(jnp.int32, 16)
    even = (iota & 1) == 0

    @plsc.parallel_loop(0, nv // 16, unroll=1, carry=acc)
    def body(t, acc):
        b = t * 16
        w = [_ld(xbuf, b + k) for k in range(16)]
        w = _aligned_net(w)
        w = [_vs(v) for v in w]
        for k in range(16):
            j = (b + k) * 16 + iota
            ia = j >> 1
            ib = jnp.where(even, jnp.maximum(ia - 1, 0),
                           jnp.minimum(ia + 1, m - 1))
            ya = plsc.load_gather(ybuf, [yoff * 16 + ia])
            yb = plsc.load_gather(ybuf, [yoff * 16 + ib])
            d = w[k] - (0.75 * ya + 0.25 * yb)
            acc = acc + d * d
        return acc
    return body


def _sort_ref(buf, n, off):
    nv = n // 16
    _mega_pass(buf, nv, off)
    for lev in range(10, n.bit_length()):
        _first_pass(buf, nv, lev, off)
        _upper_pass(buf, nv, 1 << (lev - 9), off)
        _low_pass(buf, nv, off)


def _make_sc(n, m, ncols):
    cpw = ncols // _NW
    nvx, nvy = n // 16, m // 16
    mesh = plsc.VectorSubcoreMesh(core_axis_name="c", subcore_axis_name="s",
                                  num_cores=_NC, num_subcores=_NS)

    @functools.partial(
        pl.kernel,
        out_type=jax.ShapeDtypeStruct((_NW, _LANES), jnp.float32),
        mesh=mesh,
        scratch_types=[pltpu.VMEM((n,), jnp.float32),
                       pltpu.VMEM((2 * m,), jnp.float32),
                       pltpu.VMEM((_LANES,), jnp.float32),
                       pltpu.SemaphoreType.DMA,
                       pltpu.SemaphoreType.DMA],
        compiler_params=pltpu.CompilerParams(needs_layout_passes=False),
    )
    def sc(xp_hbm, yp_hbm, out_hbm, xbuf, ybuf, obuf, ysem, xsem):
        wid = lax.axis_index("s") * _NC + lax.axis_index("c")
        col0 = wid * cpw
        pltpu.async_copy(yp_hbm.at[col0], ybuf.at[pl.ds(0, m)], ysem)

        def col_body(c, acc):
            col = col0 + c
            yoff = (c & 1) * nvy
            ybase = yoff * 16
            pltpu.async_copy(xp_hbm.at[col], xbuf, xsem)
            pltpu.make_async_copy(yp_hbm.at[col],
                                  ybuf.at[pl.ds(ybase, m)], ysem).wait()

            @pl.when(c + 1 < cpw)
            def _():
                pltpu.async_copy(yp_hbm.at[col + 1],
                                 ybuf.at[pl.ds(m - ybase, m)],
                                 ysem)

            _sort_ref(ybuf, m, yoff)
            pltpu.make_async_copy(xp_hbm.at[col], xbuf, xsem).wait()
            _mega_pass(xbuf, nvx, 0)
            for lev in range(10, n.bit_length()):
                _first_pass(xbuf, nvx, lev, 0)
                _upper_pass(xbuf, nvx, 1 << (lev - 9), 0)
                if lev < n.bit_length() - 1:
                    _low_pass(xbuf, nvx, 0)
            return _low_pass_interp(xbuf, ybuf, nvx, m, yoff, acc)

        acc = lax.fori_loop(0, cpw, col_body, jnp.zeros((16,), jnp.float32))
        obuf[...] = acc
        pltpu.sync_copy(obuf, out_hbm.at[wid])

    return sc


def kernel(x, y):
    n, d = x.shape
    m = y.shape[0]
    assert n == 2 * m, "kernel specialized to N == 2*M"
    theta = jax.random.normal(jax.random.key(42), (d, _N_PROJ), dtype=x.dtype)
    th_t = theta.T
    xp, yp = _project(x, y, th_t)
    parts = _make_sc(n, m, _N_PROJ)(xp, yp)
    swd2 = jnp.sum(parts) / (n * _N_PROJ)
    return jnp.maximum(jnp.sqrt(swd2), jnp.float32(1e-8))
